# baseline (speedup 1.0000x reference)
import jax
import jax.numpy as jnp
from jax import lax
from jax.experimental import pallas as pl
from jax.experimental.pallas import tpu as pltpu
from jax.experimental.pallas import tpu_sc as plsc

_NV = 10000
_NF = 10000
_E = 320000
_D = 128
_NG = 8

_NCORES = 2
_NSUB = 16
_C = 125
_EPT = _E // _NSUB
_NCHUNK = _EPT // _C
_ROWS = _E // _C
_HALF = _NF // 2
_TRASH = 504
_ACC_N = 5504
_IPT = _ACC_N // _NSUB
_OPT = 312
_NC2 = _NCHUNK // 2


def _edge_pass_body(tab_a_pad, tab_b, gidx_b, sidx, zeros_hbm,
                    out_hbm,
                    gidx_b_v, sidx_v, buf_a0, buf_a1, buf_m0, buf_m1,
                    acc, sem_g, sem_s):
    c = lax.axis_index("c")
    s = lax.axis_index("s")
    pltpu.sync_copy(zeros_hbm, acc.at[pl.ds(s * _IPT, _IPT)])
    plsc.subcore_barrier()

    tab_ac = tab_a_pad.at[c]
    buf_a = (buf_a0, buf_a1)
    buf_m = (buf_m0, buf_m1)

    def gathers(j, q):
        pltpu.async_copy(tab_ac.at[sidx_v.at[j]], buf_a[q], sem_g)
        pltpu.async_copy(tab_b.at[gidx_b_v.at[j]], buf_m[q], sem_g)

    def wait_gathers(j, q):
        pltpu.make_async_copy(tab_ac.at[sidx_v.at[j]], buf_a[q],
                              sem_g).wait()
        pltpu.make_async_copy(tab_b.at[gidx_b_v.at[j]], buf_m[q],
                              sem_g).wait()

    def wait_scatter(j, q):
        pltpu.make_async_copy(buf_m[q], acc.at[sidx_v.at[j]], sem_s).wait()

    def pair(jj, carry):
        for p in range(2):
            j = 2 * jj + p
            q = 1 - p
            wait_gathers(j, p)
            if p == 0:
                @pl.when(jj > 0)
                def _():
                    wait_scatter(j - 1, q)

                gathers(j + 1, q)
            else:
                @pl.when(jj < _NC2 // 2 - 1)
                def _():
                    wait_scatter(j - 1, q)
                    gathers(j + 1, q)

            ba = buf_a[p]
            bm = buf_m[p]

            def row(r5, rc):
                for u in range(5):
                    r = r5 * 5 + u
                    for k in range(_D // 16):
                        sl = pl.ds(k * 16, 16)
                        bm[r, sl] = jnp.maximum(ba[r, sl] + bm[r, sl], 0.0)
                return rc

            lax.fori_loop(0, _C // 5, row, 0)
            pltpu.async_copy(buf_m[p], acc.at[sidx_v.at[j]], sem_s,
                             add=True)
        return carry

    for h in range(2):
        base = h * _NC2
        pltpu.sync_copy(gidx_b.at[pl.ds(s * _NCHUNK + base, _NC2)], gidx_b_v)
        pltpu.sync_copy(
            sidx.at[pl.ds(c * _ROWS + s * _NCHUNK + base, _NC2)], sidx_v)
        gathers(0, 0)
        lax.fori_loop(0, _NC2 // 2, pair, 0)
        wait_scatter(_NC2 - 2, 0)
        wait_scatter(_NC2 - 1, 1)
    plsc.subcore_barrier()
    pltpu.sync_copy(acc.at[pl.ds(s * _OPT, _OPT)],
                    out_hbm.at[pl.ds(c * _HALF + s * _OPT, _OPT)])

    @pl.when(s == 0)
    def _():
        tail = _HALF - _NSUB * _OPT
        pltpu.sync_copy(acc.at[pl.ds(_NSUB * _OPT, tail)],
                        out_hbm.at[pl.ds(c * _HALF + _NSUB * _OPT, tail)])


@jax.jit
def _edge_pass(tab_a_pad, tab_b, gidx_b, sidx, zeros):
    mesh = plsc.VectorSubcoreMesh(core_axis_name="c", subcore_axis_name="s")
    return pl.kernel(
        _edge_pass_body,
        out_type=jax.ShapeDtypeStruct((_NF, _D), jnp.float32),
        mesh=mesh,
        scratch_types=[
            pltpu.VMEM((_NC2, _C), jnp.int32),
            pltpu.VMEM((_NC2, _C), jnp.int32),
            pltpu.VMEM((_C, _D), jnp.float32),
            pltpu.VMEM((_C, _D), jnp.float32),
            pltpu.VMEM((_C, _D), jnp.float32),
            pltpu.VMEM((_C, _D), jnp.float32),
            pltpu.VMEM_SHARED((_ACC_N, _D), jnp.float32),
            pltpu.SemaphoreType.DMA,
            pltpu.SemaphoreType.DMA,
        ],
    )(tab_a_pad, tab_b, gidx_b, sidx, zeros)


_R = 1000


def _row_spec():
    return pl.BlockSpec((_R, _D), lambda i: (i, 0))


_W_SPEC = pl.BlockSpec((_D, _D), lambda i: (0, 0))
_B_SPEC = pl.BlockSpec((1, _D), lambda i: (0, 0))


def _dot(x, w):
    return jnp.dot(x, w, preferred_element_type=jnp.float32)


def _msg_prep_body(f_ref, v_ref, wa, ba, wb, wc, bc, p1, q1, p2v):
    f = f_ref[...]
    v = v_ref[...]
    p1[...] = _dot(f, wa[...]) + ba[...]
    q1[...] = _dot(v, wb[...])
    p2v[...] = _dot(v, wc[...]) + bc[...]


@jax.jit
def _msg_prep(factors, variables, wa, ba, wb, wc, bc):
    return pl.pallas_call(
        _msg_prep_body,
        grid=(_NF // _R,),
        in_specs=[_row_spec(), _row_spec(), _W_SPEC, _B_SPEC, _W_SPEC,
                  _W_SPEC, _B_SPEC],
        out_specs=[_row_spec()] * 3,
        out_shape=[jax.ShapeDtypeStruct((_NF, _D), jnp.float32)] * 3,
    )(factors, variables, wa, ba, wb, wc, bc)


def _comb_f_body(f_ref, a_ref, w1, w2, b1, w3, new_f, q2f):
    f = f_ref[...]
    nf = jnp.maximum(
        _dot(f, w1[...]) + _dot(a_ref[...], w2[...]) + b1[...], 0.0)
    new_f[...] = nf
    q2f[...] = _dot(nf, w3[...])


@jax.jit
def _comb_f(factors, aggr, w1, w2, b1, w3):
    return pl.pallas_call(
        _comb_f_body,
        grid=(_NF // _R,),
        in_specs=[_row_spec(), _row_spec(), _W_SPEC, _W_SPEC, _B_SPEC,
                  _W_SPEC],
        out_specs=[_row_spec(), _row_spec()],
        out_shape=[jax.ShapeDtypeStruct((_NF, _D), jnp.float32)] * 2,
    )(factors, aggr, w1, w2, b1, w3)


def _comb_v_body(v_ref, a_ref, nf_ref, w1, w2, b1,
                 wm_f, bm_f, wm_v, wn_v, bn_v,
                 new_v, p1n, q1n, p2vn):
    v = v_ref[...]
    nv = v + jnp.maximum(
        _dot(v, w1[...]) + _dot(a_ref[...], w2[...]) + b1[...], 0.0)
    new_v[...] = nv
    p1n[...] = _dot(nf_ref[...], wm_f[...]) + bm_f[...]
    q1n[...] = _dot(nv, wm_v[...])
    p2vn[...] = _dot(nv, wn_v[...]) + bn_v[...]


@jax.jit
def _comb_v(variables, aggr, new_f, w1, w2, b1, wm_f, bm_f, wm_v,
            wn_v, bn_v):
    return pl.pallas_call(
        _comb_v_body,
        grid=(_NV // _R,),
        in_specs=[_row_spec(), _row_spec(), _row_spec(), _W_SPEC, _W_SPEC,
                  _B_SPEC, _W_SPEC, _B_SPEC, _W_SPEC, _W_SPEC, _B_SPEC],
        out_specs=[_row_spec()] * 4,
        out_shape=[jax.ShapeDtypeStruct((_NV, _D), jnp.float32)] * 4,
    )(variables, aggr, new_f, w1, w2, b1, wm_f, bm_f, wm_v, wn_v, bn_v)


def _comb_v_final_body(v_ref, a_ref, w1, w2, b1, new_v):
    v = v_ref[...]
    new_v[...] = v + jnp.maximum(
        _dot(v, w1[...]) + _dot(a_ref[...], w2[...]) + b1[...], 0.0)


@jax.jit
def _comb_v_final(variables, aggr, w1, w2, b1):
    return pl.pallas_call(
        _comb_v_final_body,
        grid=(_NV // _R,),
        in_specs=[_row_spec(), _row_spec(), _W_SPEC, _W_SPEC, _B_SPEC],
        out_specs=_row_spec(),
        out_shape=jax.ShapeDtypeStruct((_NV, _D), jnp.float32),
    )(variables, aggr, w1, w2, b1)


def _pool_body(f_ref, bi_ref, wg, bg, wn, bn, wgl, bgl, g_out):
    f = f_ref[...]
    bi = bi_ref[...]
    gate = _dot(f, wg[...]) + bg[...]
    onehot = (bi == lax.broadcasted_iota(jnp.int32, (1, _NG), 1))
    neg = jnp.float32(-jnp.inf)
    gmax = jnp.max(jnp.where(onehot, gate, neg), axis=0, keepdims=True)
    gmax_row = jnp.sum(jnp.where(onehot, gmax, 0.0), axis=1, keepdims=True)
    gexp = jnp.exp(gate - gmax_row)
    denom = jnp.sum(jnp.where(onehot, gexp, 0.0), axis=0, keepdims=True)
    denom_row = jnp.sum(jnp.where(onehot, denom, 0.0), axis=1, keepdims=True)
    attn = gexp / denom_row
    val = _dot(f, wn[...]) + bn[...]
    weighted = attn * val
    g = lax.dot_general(onehot.astype(jnp.float32), weighted,
                        (((0,), (0,)), ((), ())),
                        preferred_element_type=jnp.float32)
    g_out[...] = jnp.maximum(_dot(g, wgl[...]) + bgl[...], 0.0)


@jax.jit
def _pool(factors, bi2d, wg, bg, wn, bn, wgl, bgl):
    return pl.pallas_call(
        _pool_body,
        out_shape=jax.ShapeDtypeStruct((_NG, _D), jnp.float32),
    )(factors, bi2d, wg, bg, wn, bn, wgl, bgl)


def _pad_table(t):
    return jnp.pad(t.reshape(2, _HALF, _D),
                   ((0, 0), (0, _ACC_N - _HALF), (0, 0)))


def _scatter_lists(idx):
    trash = _HALF + (jnp.arange(_E, dtype=jnp.int32) % _TRASH)
    lo = jnp.where(idx < _HALF, idx, trash)
    hi = jnp.where(idx >= _HALF, idx - _HALF, trash)
    return jnp.concatenate([lo, hi]).reshape(2 * _ROWS, _C)


def kernel(variables, factors, edge_index, edge_attr, batch_idx, params):
    lp1, lp2 = params['layers']
    src = edge_index[0]
    dst = edge_index[1]
    src2d = src.reshape(_ROWS, _C)
    dst2d = dst.reshape(_ROWS, _C)
    sdst = _scatter_lists(dst)
    ssrc = _scatter_lists(src)
    zeros = jnp.zeros((_IPT, _D), jnp.float32)

    def halves(wb):
        w, b = wb
        return w[:_D], w[_D:], b.reshape(1, _D)

    m1f, m1v, m1b = halves(lp1['v2f_msg'])
    c1f, c1a, c1b = halves(lp1['v2f_comb'])
    n1v, n1f, n1b = halves(lp1['f2v_msg'])
    d1v, d1a, d1b = halves(lp1['f2v_comb'])
    m2f, m2v, m2b = halves(lp2['v2f_msg'])
    c2f, c2a, c2b = halves(lp2['v2f_comb'])
    n2v, n2f, n2b = halves(lp2['f2v_msg'])
    d2v, d2a, d2b = halves(lp2['f2v_comb'])

    p1, q1, p2v = _msg_prep(factors, variables, m1f, m1b, m1v, n1v, n1b)
    aggr_f = _edge_pass(_pad_table(p1), q1, src2d, sdst, zeros)
    new_f, q2f = _comb_f(factors, aggr_f, c1f, c1a, c1b, n1f)
    aggr_v = _edge_pass(_pad_table(p2v), q2f, dst2d, ssrc, zeros)
    new_v, p1n, q1n, p2vn = _comb_v(
        variables, aggr_v, new_f, d1v, d1a, d1b, m2f, m2b, m2v, n2v, n2b)
    aggr_f = _edge_pass(_pad_table(p1n), q1n, src2d, sdst, zeros)
    new_f2, q2f2 = _comb_f(new_f, aggr_f, c2f, c2a, c2b, n2f)
    aggr_v = _edge_pass(_pad_table(p2vn), q2f2, dst2d, ssrc, zeros)
    new_v2 = _comb_v_final(new_v, aggr_v, d2v, d2a, d2b)

    wg, bg = params['gate']
    wn, bn = params['att_nn']
    wgl, bgl = params['glin']
    bi2d = batch_idx.reshape(_NF, 1)
    g = _pool(new_f2, bi2d, wg, bg.reshape(1, 1), wn, bn.reshape(1, _D),
              wgl[:_D], bgl.reshape(1, _D))
    return (new_v2, new_f2, g)


# speedup vs baseline: 1.0585x; 1.0585x over previous
import jax
import jax.numpy as jnp
from jax import lax
from jax.experimental import pallas as pl
from jax.experimental.pallas import tpu as pltpu
from jax.experimental.pallas import tpu_sc as plsc

_NV = 10000
_NF = 10000
_E = 320000
_D = 128
_NG = 8

_NCORES = 2
_NSUB = 16
_C = 125
_EPT = _E // _NSUB
_NCHUNK = _EPT // _C
_ROWS = _E // _C
_HALF = _NF // 2
_TRASH = 504
_ACC_N = 5504
_IPT = _ACC_N // _NSUB
_OPT = 312
_NC2 = _NCHUNK // 2


def _edge_pass_body(tab_a_pad, tab_b, gidx_b, sidx, zeros_hbm,
                    out_hbm,
                    gidx_b_v, sidx_v, buf_a0, buf_a1, buf_m0, buf_m1,
                    acc, sem_g0, sem_g1, sem_s):
    c = lax.axis_index("c")
    s = lax.axis_index("s")
    pltpu.sync_copy(zeros_hbm, acc.at[pl.ds(s * _IPT, _IPT)])
    plsc.subcore_barrier()

    tab_ac = tab_a_pad.at[c]
    buf_a = (buf_a0, buf_a1)
    buf_m = (buf_m0, buf_m1)
    sem_g = (sem_g0, sem_g1)

    def gathers(j, q):
        pltpu.async_copy(tab_ac.at[sidx_v.at[j]], buf_a[q], sem_g[q])
        pltpu.async_copy(tab_b.at[gidx_b_v.at[j]], buf_m[q], sem_g[q])

    def wait_gathers(j, q):
        pltpu.make_async_copy(tab_ac.at[sidx_v.at[j]], buf_a[q],
                              sem_g[q]).wait()
        pltpu.make_async_copy(tab_b.at[gidx_b_v.at[j]], buf_m[q],
                              sem_g[q]).wait()

    def wait_scatter(j, q):
        pltpu.make_async_copy(buf_m[q], acc.at[sidx_v.at[j]], sem_s).wait()

    def pair(jj, carry):
        for p in range(2):
            j = 2 * jj + p
            q = 1 - p
            if p == 0:
                @pl.when(jj > 0)
                def _():
                    wait_scatter(j - 1, q)

                gathers(j + 1, q)
            else:
                @pl.when(jj < _NC2 // 2 - 1)
                def _():
                    wait_scatter(j - 1, q)
                    gathers(j + 1, q)

            wait_gathers(j, p)
            ba = buf_a[p]
            bm = buf_m[p]

            def row(r5, rc):
                for u in range(5):
                    r = r5 * 5 + u
                    for k in range(_D // 16):
                        sl = pl.ds(k * 16, 16)
                        bm[r, sl] = jnp.maximum(ba[r, sl] + bm[r, sl], 0.0)
                return rc

            lax.fori_loop(0, _C // 5, row, 0)
            pltpu.async_copy(buf_m[p], acc.at[sidx_v.at[j]], sem_s,
                             add=True)
        return carry

    for h in range(2):
        base = h * _NC2
        pltpu.sync_copy(gidx_b.at[pl.ds(s * _NCHUNK + base, _NC2)], gidx_b_v)
        pltpu.sync_copy(
            sidx.at[pl.ds(c * _ROWS + s * _NCHUNK + base, _NC2)], sidx_v)
        gathers(0, 0)
        lax.fori_loop(0, _NC2 // 2, pair, 0)
        wait_scatter(_NC2 - 2, 0)
        wait_scatter(_NC2 - 1, 1)
    plsc.subcore_barrier()
    pltpu.sync_copy(acc.at[pl.ds(s * _OPT, _OPT)],
                    out_hbm.at[pl.ds(c * _HALF + s * _OPT, _OPT)])

    @pl.when(s == 0)
    def _():
        tail = _HALF - _NSUB * _OPT
        pltpu.sync_copy(acc.at[pl.ds(_NSUB * _OPT, tail)],
                        out_hbm.at[pl.ds(c * _HALF + _NSUB * _OPT, tail)])


@jax.jit
def _edge_pass(tab_a_pad, tab_b, gidx_b, sidx, zeros):
    mesh = plsc.VectorSubcoreMesh(core_axis_name="c", subcore_axis_name="s")
    return pl.kernel(
        _edge_pass_body,
        out_type=jax.ShapeDtypeStruct((_NF, _D), jnp.float32),
        mesh=mesh,
        scratch_types=[
            pltpu.VMEM((_NC2, _C), jnp.int32),
            pltpu.VMEM((_NC2, _C), jnp.int32),
            pltpu.VMEM((_C, _D), jnp.float32),
            pltpu.VMEM((_C, _D), jnp.float32),
            pltpu.VMEM((_C, _D), jnp.float32),
            pltpu.VMEM((_C, _D), jnp.float32),
            pltpu.VMEM_SHARED((_ACC_N, _D), jnp.float32),
            pltpu.SemaphoreType.DMA,
            pltpu.SemaphoreType.DMA,
            pltpu.SemaphoreType.DMA,
        ],
    )(tab_a_pad, tab_b, gidx_b, sidx, zeros)


_R = 1000


def _row_spec():
    return pl.BlockSpec((_R, _D), lambda i: (i, 0))


_W_SPEC = pl.BlockSpec((_D, _D), lambda i: (0, 0))
_B_SPEC = pl.BlockSpec((1, _D), lambda i: (0, 0))


def _dot(x, w):
    return jnp.dot(x, w, preferred_element_type=jnp.float32)


def _msg_prep_body(f_ref, v_ref, wa, ba, wb, wc, bc, p1, q1, p2v):
    f = f_ref[...]
    v = v_ref[...]
    p1[...] = _dot(f, wa[...]) + ba[...]
    q1[...] = _dot(v, wb[...])
    p2v[...] = _dot(v, wc[...]) + bc[...]


@jax.jit
def _msg_prep(factors, variables, wa, ba, wb, wc, bc):
    return pl.pallas_call(
        _msg_prep_body,
        grid=(_NF // _R,),
        in_specs=[_row_spec(), _row_spec(), _W_SPEC, _B_SPEC, _W_SPEC,
                  _W_SPEC, _B_SPEC],
        out_specs=[_row_spec()] * 3,
        out_shape=[jax.ShapeDtypeStruct((_NF, _D), jnp.float32)] * 3,
    )(factors, variables, wa, ba, wb, wc, bc)


def _comb_f_body(f_ref, a_ref, w1, w2, b1, w3, new_f, q2f):
    f = f_ref[...]
    nf = jnp.maximum(
        _dot(f, w1[...]) + _dot(a_ref[...], w2[...]) + b1[...], 0.0)
    new_f[...] = nf
    q2f[...] = _dot(nf, w3[...])


@jax.jit
def _comb_f(factors, aggr, w1, w2, b1, w3):
    return pl.pallas_call(
        _comb_f_body,
        grid=(_NF // _R,),
        in_specs=[_row_spec(), _row_spec(), _W_SPEC, _W_SPEC, _B_SPEC,
                  _W_SPEC],
        out_specs=[_row_spec(), _row_spec()],
        out_shape=[jax.ShapeDtypeStruct((_NF, _D), jnp.float32)] * 2,
    )(factors, aggr, w1, w2, b1, w3)


def _comb_v_body(v_ref, a_ref, nf_ref, w1, w2, b1,
                 wm_f, bm_f, wm_v, wn_v, bn_v,
                 new_v, p1n, q1n, p2vn):
    v = v_ref[...]
    nv = v + jnp.maximum(
        _dot(v, w1[...]) + _dot(a_ref[...], w2[...]) + b1[...], 0.0)
    new_v[...] = nv
    p1n[...] = _dot(nf_ref[...], wm_f[...]) + bm_f[...]
    q1n[...] = _dot(nv, wm_v[...])
    p2vn[...] = _dot(nv, wn_v[...]) + bn_v[...]


@jax.jit
def _comb_v(variables, aggr, new_f, w1, w2, b1, wm_f, bm_f, wm_v,
            wn_v, bn_v):
    return pl.pallas_call(
        _comb_v_body,
        grid=(_NV // _R,),
        in_specs=[_row_spec(), _row_spec(), _row_spec(), _W_SPEC, _W_SPEC,
                  _B_SPEC, _W_SPEC, _B_SPEC, _W_SPEC, _W_SPEC, _B_SPEC],
        out_specs=[_row_spec()] * 4,
        out_shape=[jax.ShapeDtypeStruct((_NV, _D), jnp.float32)] * 4,
    )(variables, aggr, new_f, w1, w2, b1, wm_f, bm_f, wm_v, wn_v, bn_v)


def _comb_v_final_body(v_ref, a_ref, w1, w2, b1, new_v):
    v = v_ref[...]
    new_v[...] = v + jnp.maximum(
        _dot(v, w1[...]) + _dot(a_ref[...], w2[...]) + b1[...], 0.0)


@jax.jit
def _comb_v_final(variables, aggr, w1, w2, b1):
    return pl.pallas_call(
        _comb_v_final_body,
        grid=(_NV // _R,),
        in_specs=[_row_spec(), _row_spec(), _W_SPEC, _W_SPEC, _B_SPEC],
        out_specs=_row_spec(),
        out_shape=jax.ShapeDtypeStruct((_NV, _D), jnp.float32),
    )(variables, aggr, w1, w2, b1)


def _pool_body(f_ref, bi_ref, wg, bg, wn, bn, wgl, bgl, g_out):
    f = f_ref[...]
    bi = bi_ref[...]
    gate = _dot(f, wg[...]) + bg[...]
    onehot = (bi == lax.broadcasted_iota(jnp.int32, (1, _NG), 1))
    neg = jnp.float32(-jnp.inf)
    gmax = jnp.max(jnp.where(onehot, gate, neg), axis=0, keepdims=True)
    gmax_row = jnp.sum(jnp.where(onehot, gmax, 0.0), axis=1, keepdims=True)
    gexp = jnp.exp(gate - gmax_row)
    denom = jnp.sum(jnp.where(onehot, gexp, 0.0), axis=0, keepdims=True)
    denom_row = jnp.sum(jnp.where(onehot, denom, 0.0), axis=1, keepdims=True)
    attn = gexp / denom_row
    val = _dot(f, wn[...]) + bn[...]
    weighted = attn * val
    g = lax.dot_general(onehot.astype(jnp.float32), weighted,
                        (((0,), (0,)), ((), ())),
                        preferred_element_type=jnp.float32)
    g_out[...] = jnp.maximum(_dot(g, wgl[...]) + bgl[...], 0.0)


@jax.jit
def _pool(factors, bi2d, wg, bg, wn, bn, wgl, bgl):
    return pl.pallas_call(
        _pool_body,
        out_shape=jax.ShapeDtypeStruct((_NG, _D), jnp.float32),
    )(factors, bi2d, wg, bg, wn, bn, wgl, bgl)


def _pad_table(t):
    return jnp.pad(t.reshape(2, _HALF, _D),
                   ((0, 0), (0, _ACC_N - _HALF), (0, 0)))


def _scatter_lists(idx):
    trash = _HALF + (jnp.arange(_E, dtype=jnp.int32) % _TRASH)
    lo = jnp.where(idx < _HALF, idx, trash)
    hi = jnp.where(idx >= _HALF, idx - _HALF, trash)
    return jnp.concatenate([lo, hi]).reshape(2 * _ROWS, _C)


def kernel(variables, factors, edge_index, edge_attr, batch_idx, params):
    lp1, lp2 = params['layers']
    src = edge_index[0]
    dst = edge_index[1]
    src2d = src.reshape(_ROWS, _C)
    dst2d = dst.reshape(_ROWS, _C)
    sdst = _scatter_lists(dst)
    ssrc = _scatter_lists(src)
    zeros = jnp.zeros((_IPT, _D), jnp.float32)

    def halves(wb):
        w, b = wb
        return w[:_D], w[_D:], b.reshape(1, _D)

    m1f, m1v, m1b = halves(lp1['v2f_msg'])
    c1f, c1a, c1b = halves(lp1['v2f_comb'])
    n1v, n1f, n1b = halves(lp1['f2v_msg'])
    d1v, d1a, d1b = halves(lp1['f2v_comb'])
    m2f, m2v, m2b = halves(lp2['v2f_msg'])
    c2f, c2a, c2b = halves(lp2['v2f_comb'])
    n2v, n2f, n2b = halves(lp2['f2v_msg'])
    d2v, d2a, d2b = halves(lp2['f2v_comb'])

    p1, q1, p2v = _msg_prep(factors, variables, m1f, m1b, m1v, n1v, n1b)
    aggr_f = _edge_pass(_pad_table(p1), q1, src2d, sdst, zeros)
    new_f, q2f = _comb_f(factors, aggr_f, c1f, c1a, c1b, n1f)
    aggr_v = _edge_pass(_pad_table(p2v), q2f, dst2d, ssrc, zeros)
    new_v, p1n, q1n, p2vn = _comb_v(
        variables, aggr_v, new_f, d1v, d1a, d1b, m2f, m2b, m2v, n2v, n2b)
    aggr_f = _edge_pass(_pad_table(p1n), q1n, src2d, sdst, zeros)
    new_f2, q2f2 = _comb_f(new_f, aggr_f, c2f, c2a, c2b, n2f)
    aggr_v = _edge_pass(_pad_table(p2vn), q2f2, dst2d, ssrc, zeros)
    new_v2 = _comb_v_final(new_v, aggr_v, d2v, d2a, d2b)

    wg, bg = params['gate']
    wn, bn = params['att_nn']
    wgl, bgl = params['glin']
    bi2d = batch_idx.reshape(_NF, 1)
    g = _pool(new_f2, bi2d, wg, bg.reshape(1, 1), wn, bn.reshape(1, _D),
              wgl[:_D], bgl.reshape(1, _D))
    return (new_v2, new_f2, g)
